# baseline (speedup 1.0000x reference)
import functools

import jax
import jax.numpy as jnp
from jax import lax
from jax.experimental import pallas as pl
from jax.experimental.pallas import tpu as pltpu
from jax.experimental.pallas import tpu_sc as plsc

N = 10000
NPAD = 10240
E = 320000
H = 16

NW = 32
CH = 125
NCH = 80
ROWS_PER_TILE = NPAD // 16

_mesh = plsc.VectorSubcoreMesh(core_axis_name="c", subcore_axis_name="s")
_sc_params = pltpu.CompilerParams(use_tc_tiling_on_sc=False)


@functools.partial(
    pl.kernel,
    out_type=jax.ShapeDtypeStruct((2, NPAD, H), jnp.float32),
    mesh=_mesh,
    compiler_params=_sc_params,
    scratch_types=[
        pltpu.VMEM((NCH, CH), jnp.int32),
        pltpu.VMEM((128,), jnp.float32),
        pltpu.VMEM((ROWS_PER_TILE,), jnp.float32),
        pltpu.VMEM((ROWS_PER_TILE, H), jnp.float32),
        pltpu.VMEM_SHARED((NPAD,), jnp.float32),
        pltpu.SemaphoreType.DMA,
    ],
)
def _sc_deg(dstm_hbm, deg_out, idx_d, ones_v, zbuf, dv16, deg_sh, sem):
    c = lax.axis_index("c")
    s = lax.axis_index("s")
    wid = c * 16 + s
    pltpu.sync_copy(dstm_hbm.at[wid], idx_d)
    for k in range(8):
        ones_v[pl.ds(16 * k, 16)] = jnp.full((16,), 1.0, jnp.float32)
    for k in range(ROWS_PER_TILE // 16):
        zbuf[pl.ds(16 * k, 16)] = jnp.zeros((16,), jnp.float32)
    sl = pl.ds(s * ROWS_PER_TILE, ROWS_PER_TILE)
    pltpu.sync_copy(zbuf, deg_sh.at[sl])
    plsc.subcore_barrier()

    ones = ones_v.at[pl.ds(0, CH)]
    DEPTH = 8

    def start(j, carry):
        pltpu.async_copy(ones, deg_sh.at[idx_d.at[j]], sem, add=True)
        return carry

    def wait_one():
        pltpu.make_async_copy(ones, deg_sh.at[idx_d.at[0]], sem).wait()

    lax.fori_loop(0, DEPTH, start, 0)

    def roll(j, carry):
        wait_one()
        return start(j, carry)

    lax.fori_loop(DEPTH, NCH, roll, 0)

    def drain(j, carry):
        wait_one()
        return carry

    lax.fori_loop(0, DEPTH, drain, 0)
    plsc.subcore_barrier()
    pltpu.sync_copy(deg_sh.at[sl], zbuf)

    def bc(g, carry):
        dvec = zbuf[pl.ds(16 * g, 16)]
        for j in range(16):
            dv16[16 * g + j, :] = jnp.broadcast_to(dvec[j], (H,))
        return carry

    lax.fori_loop(0, ROWS_PER_TILE // 16, bc, 0)
    pltpu.sync_copy(dv16, deg_out.at[c, sl])


@functools.partial(
    pl.kernel,
    out_type=jax.ShapeDtypeStruct((2, NPAD, H), jnp.float32),
    mesh=_mesh,
    compiler_params=_sc_params,
    scratch_types=[
        pltpu.VMEM((NCH, CH), jnp.int32),
        pltpu.VMEM((NCH, CH), jnp.int32),
        pltpu.VMEM((CH, H), jnp.float32),
        pltpu.VMEM((CH, H), jnp.float32),
        pltpu.VMEM((CH, H), jnp.float32),
        pltpu.VMEM((CH, H), jnp.float32),
        pltpu.VMEM_SHARED((NPAD, H), jnp.float32),
        pltpu.SemaphoreType.DMA,
        pltpu.SemaphoreType.DMA,
        pltpu.SemaphoreType.DMA,
        pltpu.SemaphoreType.DMA,
        pltpu.SemaphoreType.DMA,
        pltpu.SemaphoreType.DMA,
        pltpu.SemaphoreType.DMA,
        pltpu.SemaphoreType.DMA,
    ],
)
def _sc_agg(tab_hbm, srcm_hbm, dstm_hbm, zeros_hbm, out_hbm,
            idx_s, idx_d, b0, b1, b2, b3, acc_sh,
            g0, g1, g2, g3, s0, s1, s2, s3):
    c = lax.axis_index("c")
    s = lax.axis_index("s")
    wid = c * 16 + s
    sl = pl.ds(s * ROWS_PER_TILE, ROWS_PER_TILE)
    pltpu.sync_copy(srcm_hbm.at[wid], idx_s)
    pltpu.sync_copy(dstm_hbm.at[wid], idx_d)
    pltpu.sync_copy(zeros_hbm.at[sl], acc_sh.at[sl])
    plsc.subcore_barrier()

    bufs = (b0, b1, b2, b3)
    gsems = (g0, g1, g2, g3)
    ssems = (s0, s1, s2, s3)

    def g_start(j, k):
        pltpu.async_copy(tab_hbm.at[idx_s.at[j]], bufs[k], gsems[k])

    def g_wait(k):
        pltpu.make_async_copy(tab_hbm.at[idx_s.at[0]], bufs[k], gsems[k]).wait()

    def s_start(j, k):
        pltpu.async_copy(bufs[k], acc_sh.at[idx_d.at[j]], ssems[k], add=True)

    def s_wait(k):
        pltpu.make_async_copy(bufs[k], acc_sh.at[idx_d.at[0]], ssems[k]).wait()

    g_start(0, 0)
    g_start(1, 1)
    g_wait(0); s_start(0, 0); g_start(2, 2)
    g_wait(1); s_start(1, 1); g_start(3, 3)

    def body(i, carry):
        j = 4 * i + 2
        for cc in range(4):
            k = (2 + cc) % 4
            g_wait(k)
            s_start(j + cc, k)
            s_wait(cc)
            g_start(j + cc + 2, cc)
        return carry

    lax.fori_loop(0, (NCH - 4) // 4, body, 0)
    g_wait(2); s_start(NCH - 2, 2)
    g_wait(3); s_start(NCH - 1, 3)
    s_wait(0); s_wait(1); s_wait(2); s_wait(3)
    plsc.subcore_barrier()
    pltpu.sync_copy(acc_sh.at[sl], out_hbm.at[c, sl])


NR = NPAD // 8
NRV = N // 8

def _tc1a_body(x8_ref, w1b_ref, h1_ref):
    h1_ref[0:NRV, :] = jnp.dot(x8_ref[...], w1b_ref[...],
                               preferred_element_type=jnp.float32)
    h1_ref[NRV:NR, :] = jnp.zeros((NR - NRV, 128), jnp.float32)


def _tc1b_body(h1_ref, degp_ref, h1s_ref, dis_ref):
    dis = lax.rsqrt(degp_ref[0] + degp_ref[1] + 1.0)
    dis_ref[...] = dis
    h1s_ref[...] = h1_ref[...] * dis


def _tc2_body(pp_ref, h1s_ref, dis_ref, b1_ref, w2b_ref, h2s_ref):
    acc = pp_ref[0] + pp_ref[1] + h1s_ref[...]
    r = jnp.maximum(acc * dis_ref[...] + b1_ref[...], 0.0)
    h2 = jnp.dot(r, w2b_ref[...], preferred_element_type=jnp.float32)
    h2s_ref[0:NRV, :] = (h2 * dis_ref[...])[0:NRV]
    h2s_ref[NRV:NR, :] = jnp.zeros((NR - NRV, 128), jnp.float32)


def _tc3_body(qp_ref, h2s_ref, dis_ref, b2_ref, out_ref):
    acc = qp_ref[0, 0:NRV, :] + qp_ref[1, 0:NRV, :] + h2s_ref[0:NRV, :]
    out_ref[...] = acc * dis_ref[0:NRV, :] + b2_ref[...]


_tc1a = pl.pallas_call(
    _tc1a_body,
    out_shape=jax.ShapeDtypeStruct((NR, 128), jnp.float32),
)
_tc1b = pl.pallas_call(
    _tc1b_body,
    out_shape=[jax.ShapeDtypeStruct((NR, 128), jnp.float32),
               jax.ShapeDtypeStruct((NR, 128), jnp.float32)],
)
_tc2 = pl.pallas_call(
    _tc2_body,
    out_shape=jax.ShapeDtypeStruct((NR, 128), jnp.float32),
)
_tc3 = pl.pallas_call(
    _tc3_body,
    out_shape=jax.ShapeDtypeStruct((NRV, 128), jnp.float32),
)


def kernel(x, edge_index, W1, b1, W2, b2):
    f32 = jnp.float32
    ei32 = edge_index.astype(jnp.int32)
    dstm = ei32[1].reshape(NW, NCH, CH)
    srcm = ei32[0].reshape(NW, NCH, CH)
    zeros2d = jnp.zeros((NPAD, H), f32)
    eye8 = jnp.eye(8, dtype=f32)
    w1b = jnp.kron(eye8, W1.astype(f32))
    w2b = jnp.kron(eye8, W2.astype(f32))
    b1w = jnp.tile(b1.astype(f32), 8).reshape(1, 128)
    b2w = jnp.tile(b2.astype(f32), 8).reshape(1, 128)
    x8 = x.astype(f32).reshape(NRV, 8 * 128)

    degp = _sc_deg(dstm)
    h1 = _tc1a(x8, w1b)
    h1s, dis = _tc1b(h1, degp.reshape(2, NR, 128))
    pp = _sc_agg(h1s.reshape(NPAD, H), srcm, dstm, zeros2d)
    h2s = _tc2(pp.reshape(2, NR, 128), h1s, dis, b1w, w2b)
    qp = _sc_agg(h2s.reshape(NPAD, H), srcm, dstm, zeros2d)
    return _tc3(qp.reshape(2, NR, 128), h2s, dis, b2w).reshape(N, H)


# speedup vs baseline: 1.1461x; 1.1461x over previous
import functools

import jax
import jax.numpy as jnp
from jax import lax
from jax.experimental import pallas as pl
from jax.experimental.pallas import tpu as pltpu
from jax.experimental.pallas import tpu_sc as plsc

N = 10000
NPAD = 10240
E = 320000
H = 16

NW = 32
CH = 125
NCH = 80
ROWS_PER_TILE = NPAD // 16

_mesh = plsc.VectorSubcoreMesh(core_axis_name="c", subcore_axis_name="s")
_sc_params = pltpu.CompilerParams(use_tc_tiling_on_sc=False)


@functools.partial(
    pl.kernel,
    out_type=jax.ShapeDtypeStruct((2, NPAD, H), jnp.float32),
    mesh=_mesh,
    compiler_params=_sc_params,
    scratch_types=[
        pltpu.VMEM((NCH, CH), jnp.int32),
        pltpu.VMEM((128,), jnp.float32),
        pltpu.VMEM((ROWS_PER_TILE,), jnp.float32),
        pltpu.VMEM((ROWS_PER_TILE, H), jnp.float32),
        pltpu.VMEM_SHARED((NPAD,), jnp.float32),
        pltpu.SemaphoreType.DMA,
    ],
)
def _sc_deg(ei_hbm, deg_out, idx_d, ones_v, zbuf, dv16, deg_sh, sem):
    c = lax.axis_index("c")
    s = lax.axis_index("s")
    wid = c * 16 + s
    pltpu.sync_copy(ei_hbm.at[1, wid], idx_d)
    for k in range(8):
        ones_v[pl.ds(16 * k, 16)] = jnp.full((16,), 1.0, jnp.float32)
    for k in range(ROWS_PER_TILE // 16):
        zbuf[pl.ds(16 * k, 16)] = jnp.zeros((16,), jnp.float32)
    sl = pl.ds(s * ROWS_PER_TILE, ROWS_PER_TILE)
    pltpu.sync_copy(zbuf, deg_sh.at[sl])
    plsc.subcore_barrier()

    ones = ones_v.at[pl.ds(0, CH)]
    DEPTH = 8

    def start(j, carry):
        pltpu.async_copy(ones, deg_sh.at[idx_d.at[j]], sem, add=True)
        return carry

    def wait_one():
        pltpu.make_async_copy(ones, deg_sh.at[idx_d.at[0]], sem).wait()

    lax.fori_loop(0, DEPTH, start, 0)

    def roll(j, carry):
        wait_one()
        return start(j, carry)

    lax.fori_loop(DEPTH, NCH, roll, 0)

    def drain(j, carry):
        wait_one()
        return carry

    lax.fori_loop(0, DEPTH, drain, 0)
    plsc.subcore_barrier()
    pltpu.sync_copy(deg_sh.at[sl], zbuf)

    def bc(g, carry):
        dvec = zbuf[pl.ds(16 * g, 16)]
        for j in range(16):
            dv16[16 * g + j, :] = jnp.broadcast_to(dvec[j], (H,))
        return carry

    lax.fori_loop(0, ROWS_PER_TILE // 16, bc, 0)
    pltpu.sync_copy(dv16, deg_out.at[c, sl])


@functools.partial(
    pl.kernel,
    out_type=jax.ShapeDtypeStruct((2, NPAD, H), jnp.float32),
    mesh=_mesh,
    compiler_params=_sc_params,
    scratch_types=[
        pltpu.VMEM((NCH, CH), jnp.int32),
        pltpu.VMEM((NCH, CH), jnp.int32),
        pltpu.VMEM((CH, H), jnp.float32),
        pltpu.VMEM((CH, H), jnp.float32),
        pltpu.VMEM((CH, H), jnp.float32),
        pltpu.VMEM((CH, H), jnp.float32),
        pltpu.VMEM((ROWS_PER_TILE, H), jnp.float32),
        pltpu.VMEM_SHARED((NPAD, H), jnp.float32),
        pltpu.SemaphoreType.DMA,
        pltpu.SemaphoreType.DMA,
        pltpu.SemaphoreType.DMA,
        pltpu.SemaphoreType.DMA,
        pltpu.SemaphoreType.DMA,
        pltpu.SemaphoreType.DMA,
        pltpu.SemaphoreType.DMA,
        pltpu.SemaphoreType.DMA,
    ],
)
def _sc_agg(tab_hbm, ei_hbm, out_hbm,
            idx_s, idx_d, b0, b1, b2, b3, zacc, acc_sh,
            g0, g1, g2, g3, s0, s1, s2, s3):
    c = lax.axis_index("c")
    s = lax.axis_index("s")
    wid = c * 16 + s
    sl = pl.ds(s * ROWS_PER_TILE, ROWS_PER_TILE)
    cp_s = pltpu.async_copy(ei_hbm.at[0, wid], idx_s, g0)
    cp_d = pltpu.async_copy(ei_hbm.at[1, wid], idx_d, g1)

    def zero(i, carry):
        zacc[i, :] = jnp.zeros((H,), jnp.float32)
        return carry

    lax.fori_loop(0, ROWS_PER_TILE, zero, 0)
    pltpu.sync_copy(zacc, acc_sh.at[sl])
    cp_s.wait()
    cp_d.wait()
    plsc.subcore_barrier()

    bufs = (b0, b1, b2, b3)
    gsems = (g0, g1, g2, g3)
    ssems = (s0, s1, s2, s3)

    def g_start(j, k):
        pltpu.async_copy(tab_hbm.at[idx_s.at[j]], bufs[k], gsems[k])

    def g_wait(k):
        pltpu.make_async_copy(tab_hbm.at[idx_s.at[0]], bufs[k], gsems[k]).wait()

    def s_start(j, k):
        pltpu.async_copy(bufs[k], acc_sh.at[idx_d.at[j]], ssems[k], add=True)

    def s_wait(k):
        pltpu.make_async_copy(bufs[k], acc_sh.at[idx_d.at[0]], ssems[k]).wait()

    g_start(0, 0)
    g_start(1, 1)
    g_wait(0); s_start(0, 0); g_start(2, 2)
    g_wait(1); s_start(1, 1); g_start(3, 3)

    def body(i, carry):
        j = 4 * i + 2
        for cc in range(4):
            k = (2 + cc) % 4
            g_wait(k)
            s_start(j + cc, k)
            s_wait(cc)
            g_start(j + cc + 2, cc)
        return carry

    lax.fori_loop(0, (NCH - 4) // 4, body, 0)
    g_wait(2); s_start(NCH - 2, 2)
    g_wait(3); s_start(NCH - 1, 3)
    s_wait(0); s_wait(1); s_wait(2); s_wait(3)
    plsc.subcore_barrier()
    pltpu.sync_copy(acc_sh.at[sl], out_hbm.at[c, sl])


NR = NPAD // 8
NRV = N // 8

def _tc1a_body(x3_ref, w1_ref, h1_ref):
    for b in range(8):
        h1_ref[0:NRV, 16 * b:16 * (b + 1)] = jnp.dot(
            x3_ref[:, b, :], w1_ref[...], preferred_element_type=jnp.float32)
    h1_ref[NRV:NR, :] = jnp.zeros((NR - NRV, 128), jnp.float32)


def _tc1b_body(h1_ref, degp_ref, h1s_ref, dis_ref):
    dis = lax.rsqrt(degp_ref[0] + degp_ref[1] + 1.0)
    dis_ref[...] = dis
    h1s_ref[...] = h1_ref[...] * dis


def _tc2_body(pp_ref, h1s_ref, dis_ref, b1_ref, w2b_ref, h2s_ref):
    acc = pp_ref[0] + pp_ref[1] + h1s_ref[...]
    r = jnp.maximum(acc * dis_ref[...] + b1_ref[...], 0.0)
    h2 = jnp.dot(r, w2b_ref[...], preferred_element_type=jnp.float32)
    h2s_ref[0:NRV, :] = (h2 * dis_ref[...])[0:NRV]
    h2s_ref[NRV:NR, :] = jnp.zeros((NR - NRV, 128), jnp.float32)


def _tc3_body(qp_ref, h2s_ref, dis_ref, b2_ref, out_ref):
    acc = qp_ref[0, 0:NRV, :] + qp_ref[1, 0:NRV, :] + h2s_ref[0:NRV, :]
    out_ref[...] = acc * dis_ref[0:NRV, :] + b2_ref[...]


_tc1a = pl.pallas_call(
    _tc1a_body,
    out_shape=jax.ShapeDtypeStruct((NR, 128), jnp.float32),
)
_tc1b = pl.pallas_call(
    _tc1b_body,
    out_shape=[jax.ShapeDtypeStruct((NR, 128), jnp.float32),
               jax.ShapeDtypeStruct((NR, 128), jnp.float32)],
)
_tc2 = pl.pallas_call(
    _tc2_body,
    out_shape=jax.ShapeDtypeStruct((NR, 128), jnp.float32),
)
_tc3 = pl.pallas_call(
    _tc3_body,
    out_shape=jax.ShapeDtypeStruct((NRV, 128), jnp.float32),
)


def kernel(x, edge_index, W1, b1, W2, b2):
    f32 = jnp.float32
    ei = edge_index.astype(jnp.int32).reshape(2, NW, NCH, CH)
    w2b = jnp.kron(jnp.eye(8, dtype=f32), W2.astype(f32))
    b1w = jnp.tile(b1.astype(f32), 8).reshape(1, 128)
    b2w = jnp.tile(b2.astype(f32), 8).reshape(1, 128)
    x3 = x.astype(f32).reshape(NRV, 8, 128)

    degp = _sc_deg(ei)
    h1 = _tc1a(x3, W1.astype(f32))
    h1s, dis = _tc1b(h1, degp.reshape(2, NR, 128))
    pp = _sc_agg(h1s.reshape(NPAD, H), ei)
    h2s = _tc2(pp.reshape(2, NR, 128), h1s, dis, b1w, w2b)
    qp = _sc_agg(h2s.reshape(NPAD, H), ei)
    return _tc3(qp.reshape(2, NR, 128), h2s, dis, b2w).reshape(N, H)
